# baseline (speedup 1.0000x reference)
import functools

import numpy as np
import jax
import jax.numpy as jnp
from jax import lax
from jax.experimental import pallas as pl
from jax.experimental.pallas import tpu as pltpu
from jax.experimental.pallas import tpu_sc as plsc

N = 10000
D_FEAT = 128
HIDDEN = 256
CLASSES = 40
LAYERS = 4
ALPHA = 0.1
THETA = 0.5
E = 320000

NTILES = 16
CH = 125
EPT = E // NTILES
NCH = EPT // CH
G = 8
NGRP = NCH // G
RPT = N // NTILES
RB = 1000


def _sc_segment_sum(h, adj_r):
    mesh = plsc.VectorSubcoreMesh(core_axis_name="c", subcore_axis_name="s")

    @functools.partial(
        pl.kernel,
        mesh=mesh,
        out_type=jax.ShapeDtypeStruct((2, N, 128), jnp.float32),
        scratch_types=[
            pltpu.VMEM((2, G, CH), jnp.int32),
            pltpu.VMEM((2, G, CH), jnp.int32),
            pltpu.VMEM((2, CH, 128), jnp.float32),
            pltpu.VMEM_SHARED((N, 128), jnp.float32),
            pltpu.SemaphoreType.DMA((2,)),
            pltpu.SemaphoreType.DMA((2,)),
            pltpu.SemaphoreType.DMA((2,)),
        ],
        compiler_params=pltpu.CompilerParams(use_tc_tiling_on_sc=False),
    )
    def k(h_hbm, adj_hbm, out_hbm, src_i, dst_i, rows_v, agg_sh,
          gsem, ssem, isem):
        c = lax.axis_index("c")
        s = lax.axis_index("s")

        def gather(gg, i, b):
            pltpu.async_copy(h_hbm.at[c].at[src_i.at[gg % 2, i]],
                             rows_v.at[b], gsem.at[b])

        def gather_wait(gg, i, b):
            pltpu.make_async_copy(h_hbm.at[c].at[src_i.at[gg % 2, i]],
                                  rows_v.at[b], gsem.at[b]).wait()

        def scatter(gg, i, b):
            pltpu.async_copy(rows_v.at[b], agg_sh.at[dst_i.at[gg % 2, i]],
                             ssem.at[b], add=True)

        def scatter_wait(gg, i, b):
            pltpu.make_async_copy(rows_v.at[b], agg_sh.at[dst_i.at[gg % 2, i]],
                                  ssem.at[b]).wait()

        def idx_fetch(gg):
            pltpu.async_copy(adj_hbm.at[0, s, gg], src_i.at[gg % 2],
                             isem.at[gg % 2])
            pltpu.async_copy(adj_hbm.at[1, s, gg], dst_i.at[gg % 2],
                             isem.at[gg % 2])

        def idx_wait(gg):
            pltpu.make_async_copy(adj_hbm.at[0, s, gg], src_i.at[gg % 2],
                                  isem.at[gg % 2]).wait()
            pltpu.make_async_copy(adj_hbm.at[1, s, gg], dst_i.at[gg % 2],
                                  isem.at[gg % 2]).wait()

        pltpu.sync_copy(adj_hbm.at[0, s, 0], src_i.at[0])
        pltpu.sync_copy(adj_hbm.at[1, s, 0], dst_i.at[0])
        gather(0, 1, 1)
        z16 = jnp.zeros((16,), jnp.float32)

        def zbody(i, carry):
            for q in range(8):
                rows_v[0, i, pl.ds(q * 16, 16)] = z16
            return carry

        lax.fori_loop(0, CH, zbody, 0)
        for j in range(RPT // CH):
            pltpu.sync_copy(rows_v.at[0],
                            agg_sh.at[pl.ds(s * RPT + j * CH, CH)])
        gather(0, 0, 0)
        plsc.subcore_barrier()

        def gbody(g, carry):
            @pl.when(g + 1 < NGRP)
            def _():
                idx_fetch(g + 1)

            for i in range(G):
                b = i % 2
                gather_wait(g, i, b)
                scatter(g, i, b)
                if i == G - 2:
                    @pl.when(g + 1 < NGRP)
                    def _():
                        idx_wait(g + 1)
                scatter_wait(g, i, b)
                if i + 2 < G:
                    gather(g, i + 2, b)
                else:
                    @pl.when(g + 1 < NGRP)
                    def _():
                        gather(g + 1, i + 2 - G, b)
            return carry

        lax.fori_loop(0, NGRP, gbody, 0)
        plsc.subcore_barrier()
        pltpu.sync_copy(agg_sh.at[pl.ds(s * RPT, RPT)],
                        out_hbm.at[c, pl.ds(s * RPT, RPT)])

    return k(h, adj_r)


def _split_h(o_ref, y):
    o_ref[0] = y[:, :128]
    o_ref[1] = y[:, 128:]


def _cat_h(ref):
    return jnp.concatenate([ref[0], ref[1]], axis=1)


def _lin0(x, w, b):
    def body(x_ref, w_ref, b_ref, o_ref):
        y = jnp.dot(x_ref[...], w_ref[...], preferred_element_type=jnp.float32)
        _split_h(o_ref, jnp.maximum(y + b_ref[...], 0.0))

    return pl.pallas_call(
        body,
        grid=(N // RB,),
        in_specs=[
            pl.BlockSpec((RB, D_FEAT), lambda i: (i, 0)),
            pl.BlockSpec((D_FEAT, HIDDEN), lambda i: (0, 0)),
            pl.BlockSpec((1, HIDDEN), lambda i: (0, 0)),
        ],
        out_specs=pl.BlockSpec((2, RB, 128), lambda i: (0, i, 0)),
        out_shape=jax.ShapeDtypeStruct((2, N, 128), jnp.float32),
    )(x, w, b)


def _layer_tc(agg, h0, w, beta):
    def body(a_ref, h0_ref, w_ref, o_ref):
        hm = (1.0 - ALPHA) * _cat_h(a_ref) + ALPHA * _cat_h(h0_ref)
        y = jnp.dot(hm, w_ref[...], preferred_element_type=jnp.float32)
        _split_h(o_ref, jnp.maximum((1.0 - beta) * hm + beta * y, 0.0))

    return pl.pallas_call(
        body,
        grid=(N // RB,),
        in_specs=[
            pl.BlockSpec((2, RB, 128), lambda i: (0, i, 0)),
            pl.BlockSpec((2, RB, 128), lambda i: (0, i, 0)),
            pl.BlockSpec((HIDDEN, HIDDEN), lambda i: (0, 0)),
        ],
        out_specs=pl.BlockSpec((2, RB, 128), lambda i: (0, i, 0)),
        out_shape=jax.ShapeDtypeStruct((2, N, 128), jnp.float32),
    )(agg, h0, w)


def _last_layer_tc(agg, h0, w, beta, w1, b1):
    def body(a_ref, h0_ref, w_ref, w1_ref, b1_ref, o_ref):
        hm = (1.0 - ALPHA) * _cat_h(a_ref) + ALPHA * _cat_h(h0_ref)
        y = jnp.dot(hm, w_ref[...], preferred_element_type=jnp.float32)
        hh = jnp.maximum((1.0 - beta) * hm + beta * y, 0.0)
        z = jnp.dot(hh, w1_ref[...], preferred_element_type=jnp.float32)
        z = z + b1_ref[...]
        m = jnp.max(z, axis=1, keepdims=True)
        ls = jnp.log(jnp.sum(jnp.exp(z - m), axis=1, keepdims=True))
        o_ref[...] = z - m - ls

    return pl.pallas_call(
        body,
        grid=(N // RB,),
        in_specs=[
            pl.BlockSpec((2, RB, 128), lambda i: (0, i, 0)),
            pl.BlockSpec((2, RB, 128), lambda i: (0, i, 0)),
            pl.BlockSpec((HIDDEN, HIDDEN), lambda i: (0, 0)),
            pl.BlockSpec((HIDDEN, CLASSES), lambda i: (0, 0)),
            pl.BlockSpec((1, CLASSES), lambda i: (0, 0)),
        ],
        out_specs=pl.BlockSpec((RB, CLASSES), lambda i: (i, 0)),
        out_shape=jax.ShapeDtypeStruct((N, CLASSES), jnp.float32),
    )(agg, h0, w, w1, b1)


def kernel(adj_t, x, lin0_W, lin0_b, conv_W, lin1_W, lin1_b):
    adj_r = adj_t.astype(jnp.int32).reshape(2, NTILES, NGRP, G, CH)

    h = _lin0(x, lin0_W, lin0_b.reshape(1, HIDDEN))
    h0 = h
    for layer in range(LAYERS):
        beta = float(np.log(THETA / (layer + 1) + 1.0))
        agg = _sc_segment_sum(h, adj_r)
        if layer < LAYERS - 1:
            h = _layer_tc(agg, h0, conv_W[layer], beta)
        else:
            return _last_layer_tc(agg, h0, conv_W[layer], beta,
                                  lin1_W, lin1_b.reshape(1, CLASSES))


# speedup vs baseline: 1.0078x; 1.0078x over previous
import functools

import numpy as np
import jax
import jax.numpy as jnp
from jax import lax
from jax.experimental import pallas as pl
from jax.experimental.pallas import tpu as pltpu
from jax.experimental.pallas import tpu_sc as plsc

N = 10000
D_FEAT = 128
HIDDEN = 256
CLASSES = 40
LAYERS = 4
ALPHA = 0.1
THETA = 0.5
E = 320000

NTILES = 16
CH = 125
EPT = E // NTILES
NCH = EPT // CH
G = 8
NGRP = NCH // G
RPT = N // NTILES
RB = 2000


def _sc_segment_sum(h, adj_r):
    mesh = plsc.VectorSubcoreMesh(core_axis_name="c", subcore_axis_name="s")

    @functools.partial(
        pl.kernel,
        mesh=mesh,
        out_type=jax.ShapeDtypeStruct((2, N, 128), jnp.float32),
        scratch_types=[
            pltpu.VMEM((2, G, CH), jnp.int32),
            pltpu.VMEM((2, G, CH), jnp.int32),
            pltpu.VMEM((2, CH, 128), jnp.float32),
            pltpu.VMEM_SHARED((N, 128), jnp.float32),
            pltpu.SemaphoreType.DMA((2,)),
            pltpu.SemaphoreType.DMA((2,)),
            pltpu.SemaphoreType.DMA((2,)),
        ],
        compiler_params=pltpu.CompilerParams(use_tc_tiling_on_sc=False),
    )
    def k(h_hbm, adj_hbm, out_hbm, src_i, dst_i, rows_v, agg_sh,
          gsem, ssem, isem):
        c = lax.axis_index("c")
        s = lax.axis_index("s")

        def gather(gg, i, b):
            pltpu.async_copy(h_hbm.at[c].at[src_i.at[gg % 2, i]],
                             rows_v.at[b], gsem.at[b])

        def gather_wait(gg, i, b):
            pltpu.make_async_copy(h_hbm.at[c].at[src_i.at[gg % 2, i]],
                                  rows_v.at[b], gsem.at[b]).wait()

        def scatter(gg, i, b):
            pltpu.async_copy(rows_v.at[b], agg_sh.at[dst_i.at[gg % 2, i]],
                             ssem.at[b], add=True)

        def scatter_wait(gg, i, b):
            pltpu.make_async_copy(rows_v.at[b], agg_sh.at[dst_i.at[gg % 2, i]],
                                  ssem.at[b]).wait()

        def idx_fetch(gg):
            pltpu.async_copy(adj_hbm.at[0, s, gg], src_i.at[gg % 2],
                             isem.at[gg % 2])
            pltpu.async_copy(adj_hbm.at[1, s, gg], dst_i.at[gg % 2],
                             isem.at[gg % 2])

        def idx_wait(gg):
            pltpu.make_async_copy(adj_hbm.at[0, s, gg], src_i.at[gg % 2],
                                  isem.at[gg % 2]).wait()
            pltpu.make_async_copy(adj_hbm.at[1, s, gg], dst_i.at[gg % 2],
                                  isem.at[gg % 2]).wait()

        pltpu.sync_copy(adj_hbm.at[0, s, 0], src_i.at[0])
        pltpu.sync_copy(adj_hbm.at[1, s, 0], dst_i.at[0])
        gather(0, 1, 1)
        z16 = jnp.zeros((16,), jnp.float32)

        def zbody(i, carry):
            for q in range(8):
                rows_v[0, i, pl.ds(q * 16, 16)] = z16
            return carry

        lax.fori_loop(0, CH, zbody, 0)
        for j in range(RPT // CH):
            pltpu.sync_copy(rows_v.at[0],
                            agg_sh.at[pl.ds(s * RPT + j * CH, CH)])
        gather(0, 0, 0)
        plsc.subcore_barrier()

        def gbody(g, carry):
            @pl.when(g + 1 < NGRP)
            def _():
                idx_fetch(g + 1)

            for i in range(G):
                b = i % 2
                gather_wait(g, i, b)
                scatter(g, i, b)
                if i == G - 2:
                    @pl.when(g + 1 < NGRP)
                    def _():
                        idx_wait(g + 1)
                scatter_wait(g, i, b)
                if i + 2 < G:
                    gather(g, i + 2, b)
                else:
                    @pl.when(g + 1 < NGRP)
                    def _():
                        gather(g + 1, i + 2 - G, b)
            return carry

        lax.fori_loop(0, NGRP, gbody, 0)
        plsc.subcore_barrier()
        pltpu.sync_copy(agg_sh.at[pl.ds(s * RPT, RPT)],
                        out_hbm.at[c, pl.ds(s * RPT, RPT)])

    return k(h, adj_r)


def _split_h(o_ref, y):
    o_ref[0] = y[:, :128]
    o_ref[1] = y[:, 128:]


def _cat_h(ref):
    return jnp.concatenate([ref[0], ref[1]], axis=1)


def _lin0(x, w, b):
    def body(x_ref, w_ref, b_ref, o_ref):
        y = jnp.dot(x_ref[...], w_ref[...], preferred_element_type=jnp.float32)
        _split_h(o_ref, jnp.maximum(y + b_ref[...], 0.0))

    return pl.pallas_call(
        body,
        grid=(N // RB,),
        in_specs=[
            pl.BlockSpec((RB, D_FEAT), lambda i: (i, 0)),
            pl.BlockSpec((D_FEAT, HIDDEN), lambda i: (0, 0)),
            pl.BlockSpec((1, HIDDEN), lambda i: (0, 0)),
        ],
        out_specs=pl.BlockSpec((2, RB, 128), lambda i: (0, i, 0)),
        out_shape=jax.ShapeDtypeStruct((2, N, 128), jnp.float32),
    )(x, w, b)


def _layer_tc(agg, h0, w, beta):
    def body(a_ref, h0_ref, w_ref, o_ref):
        hm = (1.0 - ALPHA) * _cat_h(a_ref) + ALPHA * _cat_h(h0_ref)
        y = jnp.dot(hm, w_ref[...], preferred_element_type=jnp.float32)
        _split_h(o_ref, jnp.maximum((1.0 - beta) * hm + beta * y, 0.0))

    return pl.pallas_call(
        body,
        grid=(N // RB,),
        in_specs=[
            pl.BlockSpec((2, RB, 128), lambda i: (0, i, 0)),
            pl.BlockSpec((2, RB, 128), lambda i: (0, i, 0)),
            pl.BlockSpec((HIDDEN, HIDDEN), lambda i: (0, 0)),
        ],
        out_specs=pl.BlockSpec((2, RB, 128), lambda i: (0, i, 0)),
        out_shape=jax.ShapeDtypeStruct((2, N, 128), jnp.float32),
    )(agg, h0, w)


def _last_layer_tc(agg, h0, w, beta, w1, b1):
    def body(a_ref, h0_ref, w_ref, w1_ref, b1_ref, o_ref):
        hm = (1.0 - ALPHA) * _cat_h(a_ref) + ALPHA * _cat_h(h0_ref)
        y = jnp.dot(hm, w_ref[...], preferred_element_type=jnp.float32)
        hh = jnp.maximum((1.0 - beta) * hm + beta * y, 0.0)
        z = jnp.dot(hh, w1_ref[...], preferred_element_type=jnp.float32)
        z = z + b1_ref[...]
        m = jnp.max(z, axis=1, keepdims=True)
        ls = jnp.log(jnp.sum(jnp.exp(z - m), axis=1, keepdims=True))
        o_ref[...] = z - m - ls

    return pl.pallas_call(
        body,
        grid=(N // RB,),
        in_specs=[
            pl.BlockSpec((2, RB, 128), lambda i: (0, i, 0)),
            pl.BlockSpec((2, RB, 128), lambda i: (0, i, 0)),
            pl.BlockSpec((HIDDEN, HIDDEN), lambda i: (0, 0)),
            pl.BlockSpec((HIDDEN, CLASSES), lambda i: (0, 0)),
            pl.BlockSpec((1, CLASSES), lambda i: (0, 0)),
        ],
        out_specs=pl.BlockSpec((RB, CLASSES), lambda i: (i, 0)),
        out_shape=jax.ShapeDtypeStruct((N, CLASSES), jnp.float32),
    )(agg, h0, w, w1, b1)


def kernel(adj_t, x, lin0_W, lin0_b, conv_W, lin1_W, lin1_b):
    adj_r = adj_t.astype(jnp.int32).reshape(2, NTILES, NGRP, G, CH)

    h = _lin0(x, lin0_W, lin0_b.reshape(1, HIDDEN))
    h0 = h
    for layer in range(LAYERS):
        beta = float(np.log(THETA / (layer + 1) + 1.0))
        agg = _sc_segment_sum(h, adj_r)
        if layer < LAYERS - 1:
            h = _layer_tc(agg, h0, conv_W[layer], beta)
        else:
            return _last_layer_tc(agg, h0, conv_W[layer], beta,
                                  lin1_W, lin1_b.reshape(1, CLASSES))
